# baseline (speedup 1.0000x reference)
import functools

import jax
import jax.numpy as jnp
from jax import lax
from jax.experimental import pallas as pl
from jax.experimental.pallas import tpu as pltpu
from jax.experimental.pallas import tpu_sc as plsc

VOCAB = 100000
EMBED_DIM = 128
CONTEXT = 200
HIDDEN = 128
KDIM = CONTEXT * EMBED_DIM
KHALF = KDIM // 2

N_SPLIT = 4
GROUP = VOCAB // N_SPLIT
V_TILE = 1000
STEPS = GROUP // V_TILE
ROWS_PER_STEP = V_TILE
N_ROWS = VOCAB // V_TILE

ROWS_PER_WORKER = 8
N_WORKERS = CONTEXT // ROWS_PER_WORKER


def _sc_gather_kernel(idx_hbm, table_hbm, out_hbm, idx_v, rows_v, sem):
    wid = lax.axis_index("s") * 2 + lax.axis_index("c")

    @pl.when(wid < N_WORKERS)
    def _():
        base = wid * ROWS_PER_WORKER
        pltpu.sync_copy(idx_hbm.at[pl.ds(base, ROWS_PER_WORKER)], idx_v)
        pltpu.async_copy(table_hbm.at[idx_v], rows_v, sem).wait()
        pltpu.sync_copy(rows_v, out_hbm.at[pl.ds(base, ROWS_PER_WORKER)])


@jax.jit
def _sc_gather(idx, table):
    mesh = plsc.VectorSubcoreMesh(core_axis_name="c", subcore_axis_name="s")
    return pl.kernel(
        _sc_gather_kernel,
        mesh=mesh,
        out_type=jax.ShapeDtypeStruct((CONTEXT, EMBED_DIM), jnp.float32),
        scratch_types=[
            pltpu.VMEM((ROWS_PER_WORKER,), jnp.int32),
            pltpu.VMEM((ROWS_PER_WORKER, EMBED_DIM), jnp.float32),
            pltpu.SemaphoreType.DMA,
        ],
    )(idx, table)


def _mlp_kernel(emb_ref, w1a_ref, w1b_ref, b1_ref, w2a_ref, w2b_ref,
                w2c_ref, w2d_ref, b2_ref, out_ref, hid_ref):
    t = pl.program_id(0)

    @pl.when(t == 0)
    def _():
        ha = lax.dot_general(
            emb_ref[:, :KHALF], w1a_ref[...], (((1,), (1,)), ((), ())),
            preferred_element_type=jnp.float32)
        hb = lax.dot_general(
            emb_ref[:, KHALF:], w1b_ref[...], (((1,), (1,)), ((), ())),
            preferred_element_type=jnp.float32)
        hid_ref[...] = jnp.maximum(ha + hb + b1_ref[...], 0.0)

    h = hid_ref[...]
    for g, w2_ref in enumerate((w2a_ref, w2b_ref, w2c_ref, w2d_ref)):
        logits = lax.dot_general(
            h, w2_ref[...], (((1,), (1,)), ((), ())),
            preferred_element_type=jnp.float32)
        row = g * STEPS + t
        out_ref[pl.ds(row, 1), :] = logits + b2_ref[pl.ds(row, 1), :]

    @pl.when(t == STEPS - 1)
    def _():
        x = out_ref[...]
        m = jnp.max(x)
        lse = m + jnp.log(jnp.sum(jnp.exp(x - m)))
        out_ref[...] = x - lse


@jax.jit
def _tc_mlp(embflat, W1, b1, W2, b2):
    w2_specs = [
        pl.BlockSpec((V_TILE, EMBED_DIM),
                     functools.partial(lambda g, t: (g * STEPS + t, 0), g))
        for g in range(N_SPLIT)
    ]
    return pl.pallas_call(
        _mlp_kernel,
        grid=(STEPS,),
        in_specs=[
            pl.BlockSpec((1, KDIM), lambda t: (0, 0)),
            pl.BlockSpec((HIDDEN, KHALF), lambda t: (0, 0)),
            pl.BlockSpec((HIDDEN, KHALF), lambda t: (0, 1)),
            pl.BlockSpec((1, HIDDEN), lambda t: (0, 0)),
            *w2_specs,
            pl.BlockSpec((N_ROWS, V_TILE), lambda t: (0, 0)),
        ],
        out_specs=pl.BlockSpec((N_ROWS, V_TILE), lambda t: (0, 0)),
        out_shape=jax.ShapeDtypeStruct((N_ROWS, V_TILE), jnp.float32),
        scratch_shapes=[pltpu.VMEM((1, HIDDEN), jnp.float32)],
        compiler_params=pltpu.CompilerParams(
            dimension_semantics=("arbitrary",)),
    )(embflat, W1, W1, b1, W2, W2, W2, W2, b2)


def kernel(inputs, emb, W1, b1, W2, b2):
    embeds = _sc_gather(inputs.astype(jnp.int32), emb)
    embflat = embeds.reshape(1, KDIM)
    out = _tc_mlp(embflat, W1, b1.reshape(1, HIDDEN), W2,
                  b2.reshape(N_ROWS, V_TILE))
    return out.reshape(1, VOCAB)


# speedup vs baseline: 1.2689x; 1.2689x over previous
import jax
import jax.numpy as jnp
from jax import lax
from jax.experimental import pallas as pl
from jax.experimental.pallas import tpu as pltpu
from jax.experimental.pallas import tpu_sc as plsc

VOCAB = 100000
EMBED_DIM = 128
CONTEXT = 200
HIDDEN = 128
KDIM = CONTEXT * EMBED_DIM

TILE = 6400
NTILE = VOCAB // TILE
TAIL = VOCAB - NTILE * TILE
NBUF = 5

ROWS_PER_WORKER = 8
N_WORKERS = CONTEXT // ROWS_PER_WORKER


def _sc_gather_kernel(idx_hbm, table_hbm, out_hbm, idx_v, rows_v, sem):
    wid = lax.axis_index("s") * 2 + lax.axis_index("c")

    @pl.when(wid < N_WORKERS)
    def _():
        base = wid * ROWS_PER_WORKER
        pltpu.sync_copy(idx_hbm.at[pl.ds(base, ROWS_PER_WORKER)], idx_v)
        pltpu.async_copy(table_hbm.at[idx_v], rows_v, sem).wait()
        pltpu.sync_copy(rows_v, out_hbm.at[pl.ds(base, ROWS_PER_WORKER)])


@jax.jit
def _sc_gather(idx, table):
    mesh = plsc.VectorSubcoreMesh(core_axis_name="c", subcore_axis_name="s")
    return pl.kernel(
        _sc_gather_kernel,
        mesh=mesh,
        out_type=jax.ShapeDtypeStruct((CONTEXT, EMBED_DIM), jnp.float32),
        scratch_types=[
            pltpu.VMEM((ROWS_PER_WORKER,), jnp.int32),
            pltpu.VMEM((ROWS_PER_WORKER, EMBED_DIM), jnp.float32),
            pltpu.SemaphoreType.DMA,
        ],
    )(idx, table)


def _nt_dot(a, b):
    return lax.dot_general(a, b, (((1,), (1,)), ((), ())),
                           preferred_element_type=jnp.float32)


def _mlp_kernel(emb_ref, b1_ref, b2_ref, w1_hbm, w2_hbm, out_ref,
                w1_v, bufs, tail_v, sems):
    w1_cp = pltpu.make_async_copy(w1_hbm, w1_v, sems.at[NBUF])
    w1_cp.start()
    copies = [None] * NTILE
    for t in range(NBUF):
        copies[t] = pltpu.make_async_copy(
            w2_hbm.at[pl.ds(t * TILE, TILE)], bufs.at[t], sems.at[t])
        copies[t].start()
    tail_cp = pltpu.make_async_copy(
        w2_hbm.at[pl.ds(NTILE * TILE, TAIL)], tail_v, sems.at[NBUF + 1])
    tail_cp.start()

    w1_cp.wait()
    hid = jnp.maximum(_nt_dot(emb_ref[...], w1_v[...]) + b1_ref[...], 0.0)

    for t in range(NTILE):
        copies[t].wait()
        logits = _nt_dot(hid, bufs[t % NBUF])
        sl = pl.ds(t * TILE, TILE)
        out_ref[:, sl] = logits + b2_ref[:, sl]
        nt = t + NBUF
        if nt < NTILE:
            copies[nt] = pltpu.make_async_copy(
                w2_hbm.at[pl.ds(nt * TILE, TILE)], bufs.at[nt % NBUF],
                sems.at[nt % NBUF])
            copies[nt].start()

    tail_cp.wait()
    logits = _nt_dot(hid, tail_v[...])
    sl = pl.ds(NTILE * TILE, TAIL)
    out_ref[:, sl] = logits + b2_ref[:, sl]

    x = out_ref[...]
    m = jnp.max(x)
    lse = m + jnp.log(jnp.sum(jnp.exp(x - m)))
    out_ref[...] = x - lse


@jax.jit
def _tc_mlp(embflat, b1, b2, W1, W2):
    return pl.pallas_call(
        _mlp_kernel,
        in_specs=[
            pl.BlockSpec(memory_space=pltpu.MemorySpace.VMEM),
            pl.BlockSpec(memory_space=pltpu.MemorySpace.VMEM),
            pl.BlockSpec(memory_space=pltpu.MemorySpace.VMEM),
            pl.BlockSpec(memory_space=pltpu.MemorySpace.HBM),
            pl.BlockSpec(memory_space=pltpu.MemorySpace.HBM),
        ],
        out_specs=pl.BlockSpec(memory_space=pltpu.MemorySpace.VMEM),
        out_shape=jax.ShapeDtypeStruct((1, VOCAB), jnp.float32),
        scratch_shapes=[
            pltpu.VMEM((HIDDEN, KDIM), jnp.float32),
            pltpu.VMEM((NBUF, TILE, EMBED_DIM), jnp.float32),
            pltpu.VMEM((TAIL, EMBED_DIM), jnp.float32),
            pltpu.SemaphoreType.DMA((NBUF + 2,)),
        ],
    )(embflat, b1, b2, W1, W2)


def kernel(inputs, emb, W1, b1, W2, b2):
    embeds = _sc_gather(inputs.astype(jnp.int32), emb)
    embflat = embeds.reshape(1, KDIM)
    out = _tc_mlp(embflat, b1.reshape(1, HIDDEN), b2.reshape(1, VOCAB),
                  W1, W2)
    return out


# speedup vs baseline: 1.8012x; 1.4195x over previous
import jax
import jax.numpy as jnp
from jax import lax
from jax.experimental import pallas as pl
from jax.experimental.pallas import tpu as pltpu

VOCAB = 100000
EMBED_DIM = 128
CONTEXT = 200
HIDDEN = 128
KDIM = CONTEXT * EMBED_DIM

TILE = 6400
NTILE = VOCAB // TILE
TAIL = VOCAB - NTILE * TILE
NBUF = 5


def _nt_dot(a, b):
    return lax.dot_general(a, b, (((1,), (1,)), ((), ())),
                           preferred_element_type=jnp.float32)


def _mlp_kernel(idx_ref, b1_ref, b2_ref, emb_hbm, w1_hbm, w2_hbm, out_ref,
                emb_v, w1_v, bufs, tail_v, sems, gsem):
    w1_cp = pltpu.make_async_copy(w1_hbm, w1_v, sems.at[NBUF])
    w1_cp.start()
    copies = [None] * NTILE
    for t in range(NBUF):
        copies[t] = pltpu.make_async_copy(
            w2_hbm.at[pl.ds(t * TILE, TILE)], bufs.at[t], sems.at[t])
        copies[t].start()
    tail_cp = pltpu.make_async_copy(
        w2_hbm.at[pl.ds(NTILE * TILE, TAIL)], tail_v, sems.at[NBUF + 1])
    tail_cp.start()

    gcp = [None] * CONTEXT
    for c in range(CONTEXT):
        gcp[c] = pltpu.make_async_copy(
            emb_hbm.at[pl.ds(idx_ref[c], 1)],
            emb_v.at[:, pl.ds(c * EMBED_DIM, EMBED_DIM)], gsem)
        gcp[c].start()
    for c in range(CONTEXT):
        gcp[c].wait()

    w1_cp.wait()
    hid = jnp.maximum(_nt_dot(emb_v[...], w1_v[...]) + b1_ref[...], 0.0)

    for t in range(NTILE):
        copies[t].wait()
        logits = _nt_dot(hid, bufs[t % NBUF])
        sl = pl.ds(t * TILE, TILE)
        out_ref[:, sl] = logits + b2_ref[:, sl]
        nt = t + NBUF
        if nt < NTILE:
            copies[nt] = pltpu.make_async_copy(
                w2_hbm.at[pl.ds(nt * TILE, TILE)], bufs.at[nt % NBUF],
                sems.at[nt % NBUF])
            copies[nt].start()

    tail_cp.wait()
    logits = _nt_dot(hid, tail_v[...])
    sl = pl.ds(NTILE * TILE, TAIL)
    out_ref[:, sl] = logits + b2_ref[:, sl]

    x = out_ref[...]
    m = jnp.max(x)
    lse = m + jnp.log(jnp.sum(jnp.exp(x - m)))
    out_ref[...] = x - lse


@jax.jit
def _tc_mlp(idx, b1, b2, emb, W1, W2):
    return pl.pallas_call(
        _mlp_kernel,
        in_specs=[
            pl.BlockSpec(memory_space=pltpu.MemorySpace.SMEM),
            pl.BlockSpec(memory_space=pltpu.MemorySpace.VMEM),
            pl.BlockSpec(memory_space=pltpu.MemorySpace.VMEM),
            pl.BlockSpec(memory_space=pltpu.MemorySpace.HBM),
            pl.BlockSpec(memory_space=pltpu.MemorySpace.HBM),
            pl.BlockSpec(memory_space=pltpu.MemorySpace.HBM),
        ],
        out_specs=pl.BlockSpec(memory_space=pltpu.MemorySpace.VMEM),
        out_shape=jax.ShapeDtypeStruct((1, VOCAB), jnp.float32),
        scratch_shapes=[
            pltpu.VMEM((1, KDIM), jnp.float32),
            pltpu.VMEM((HIDDEN, KDIM), jnp.float32),
            pltpu.VMEM((NBUF, TILE, EMBED_DIM), jnp.float32),
            pltpu.VMEM((TAIL, EMBED_DIM), jnp.float32),
            pltpu.SemaphoreType.DMA((NBUF + 2,)),
            pltpu.SemaphoreType.DMA,
        ],
    )(idx, b1, b2, emb, W1, W2)


def kernel(inputs, emb, W1, b1, W2, b2):
    return _tc_mlp(inputs.astype(jnp.int32), b1.reshape(1, HIDDEN),
                   b2.reshape(1, VOCAB), emb, W1, W2)


# speedup vs baseline: 1.9544x; 1.0851x over previous
import jax
import jax.numpy as jnp
from jax import lax
from jax.experimental import pallas as pl
from jax.experimental.pallas import tpu as pltpu

VOCAB = 100000
EMBED_DIM = 128
CONTEXT = 200
HIDDEN = 128
KDIM = CONTEXT * EMBED_DIM

TILE = 6400
NTILE = VOCAB // TILE
TAIL = VOCAB - NTILE * TILE
NBUF = 5


def _nt_dot(a, b):
    return lax.dot_general(a, b, (((1,), (1,)), ((), ())),
                           preferred_element_type=jnp.float32)


def _mlp_kernel(idx_ref, b1_ref, b2_ref, emb_hbm, w1_hbm, w2_hbm, out_ref,
                emb_v, w1_v, bufs, tail_v, sems, gsem):
    gcp = [None] * CONTEXT
    for c in range(CONTEXT):
        gcp[c] = pltpu.make_async_copy(
            emb_hbm.at[pl.ds(idx_ref[c], 1)],
            emb_v.at[:, pl.ds(c * EMBED_DIM, EMBED_DIM)], gsem)
        gcp[c].start()

    w1_cp = pltpu.make_async_copy(w1_hbm, w1_v, sems.at[NBUF])
    w1_cp.start()
    copies = [None] * NTILE
    for t in range(NBUF):
        copies[t] = pltpu.make_async_copy(
            w2_hbm.at[pl.ds(t * TILE, TILE)], bufs.at[t], sems.at[t])
        copies[t].start()
    tail_cp = pltpu.make_async_copy(
        w2_hbm.at[pl.ds(NTILE * TILE, TAIL)], tail_v, sems.at[NBUF + 1])
    tail_cp.start()

    for c in range(CONTEXT):
        gcp[c].wait()
    w1_cp.wait()
    hid = jnp.maximum(_nt_dot(emb_v[...], w1_v[...]) + b1_ref[...], 0.0)

    for t in range(NTILE):
        copies[t].wait()
        logits = _nt_dot(hid, bufs[t % NBUF])
        sl = pl.ds(t * TILE, TILE)
        out_ref[:, sl] = logits + b2_ref[sl].reshape(1, TILE)
        nt = t + NBUF
        if nt < NTILE:
            copies[nt] = pltpu.make_async_copy(
                w2_hbm.at[pl.ds(nt * TILE, TILE)], bufs.at[nt % NBUF],
                sems.at[nt % NBUF])
            copies[nt].start()

    tail_cp.wait()
    logits = _nt_dot(hid, tail_v[...])
    sl = pl.ds(NTILE * TILE, TAIL)
    out_ref[:, sl] = logits + b2_ref[sl].reshape(1, TAIL)

    x = out_ref[...]
    m = jnp.max(x)
    lse = m + jnp.log(jnp.sum(jnp.exp(x - m)))
    out_ref[...] = x - lse


@jax.jit
def _tc_mlp(idx, b1, b2, emb, W1, W2):
    return pl.pallas_call(
        _mlp_kernel,
        in_specs=[
            pl.BlockSpec(memory_space=pltpu.MemorySpace.SMEM),
            pl.BlockSpec(memory_space=pltpu.MemorySpace.VMEM),
            pl.BlockSpec(memory_space=pltpu.MemorySpace.VMEM),
            pl.BlockSpec(memory_space=pltpu.MemorySpace.HBM),
            pl.BlockSpec(memory_space=pltpu.MemorySpace.HBM),
            pl.BlockSpec(memory_space=pltpu.MemorySpace.HBM),
        ],
        out_specs=pl.BlockSpec(memory_space=pltpu.MemorySpace.VMEM),
        out_shape=jax.ShapeDtypeStruct((1, VOCAB), jnp.float32),
        scratch_shapes=[
            pltpu.VMEM((1, KDIM), jnp.float32),
            pltpu.VMEM((HIDDEN, KDIM), jnp.float32),
            pltpu.VMEM((NBUF, TILE, EMBED_DIM), jnp.float32),
            pltpu.VMEM((TAIL, EMBED_DIM), jnp.float32),
            pltpu.SemaphoreType.DMA((NBUF + 2,)),
            pltpu.SemaphoreType.DMA,
        ],
    )(idx, b1, b2, emb, W1, W2)


def kernel(inputs, emb, W1, b1, W2, b2):
    return _tc_mlp(inputs.astype(jnp.int32), b1.reshape(1, HIDDEN),
                   b2, emb, W1, W2)
